# Initial kernel scaffold; baseline (speedup 1.0000x reference)
import jax
import jax.numpy as jnp
from jax.experimental import pallas as pl


def kernel(x, single, emb_weight):
    raise NotImplementedError("write your pallas kernel here")


# speedup vs baseline: 1.4761x; 1.4761x over previous
import functools

import jax
import jax.numpy as jnp
from jax import lax
from jax.experimental import pallas as pl
from jax.experimental.pallas import tpu as pltpu
from jax.experimental.pallas import tpu_sc as plsc

VOCAB = 1000000
EMB = 32
B = 4096
L = 200

NC = 2
NS = 16
NW = NC * NS
NB = B * L
ROWS_PER_W = NB // NW
STREAM = 128
CHUNK = 1024
S_PER_CHUNK = CHUNK // STREAM
N_CHUNK = ROWS_PER_W // CHUNK
N_STREAMS = ROWS_PER_W // STREAM

_mesh = plsc.VectorSubcoreMesh(core_axis_name="c", subcore_axis_name="s")


@functools.partial(
    pl.kernel,
    out_type=jax.ShapeDtypeStruct((NB, EMB), jnp.float32),
    mesh=_mesh,
    scratch_types=[
        pltpu.VMEM((N_STREAMS, STREAM), jnp.int32),
        pltpu.VMEM((CHUNK, EMB), jnp.float32),
        pltpu.SemaphoreType.DMA,
    ],
    compiler_params=pltpu.CompilerParams(use_tc_tiling_on_sc=False),
)
def _embed_sc(idx_hbm, table_hbm, out_hbm, idx_v, rows_v, sem):
    wid = lax.axis_index("s") * NC + lax.axis_index("c")
    base = wid * ROWS_PER_W

    pltpu.sync_copy(idx_hbm.at[wid], idx_v)

    def chunk_body(c, _):
        copies = []
        for s in range(S_PER_CHUNK):
            j = c * S_PER_CHUNK + s
            copies.append(
                pltpu.async_copy(
                    table_hbm.at[idx_v.at[j]],
                    rows_v.at[pl.ds(s * STREAM, STREAM)],
                    sem,
                )
            )
        for cp in copies:
            cp.wait()
        pltpu.sync_copy(rows_v, out_hbm.at[pl.ds(base + c * CHUNK, CHUNK)])
        return 0

    lax.fori_loop(0, N_CHUNK, chunk_body, 0)


def kernel(x, single, emb_weight):
    idx = x.reshape(NW, N_STREAMS, STREAM).astype(jnp.int32)
    out = _embed_sc(idx, emb_weight)
    return out.reshape(B, L, EMB)


# speedup vs baseline: 1.5026x; 1.0180x over previous
import functools

import jax
import jax.numpy as jnp
from jax import lax
from jax.experimental import pallas as pl
from jax.experimental.pallas import tpu as pltpu
from jax.experimental.pallas import tpu_sc as plsc

VOCAB = 1000000
EMB = 32
B = 4096
L = 200

NC = 2
NS = 16
NW = NC * NS
NB = B * L
ROWS_PER_W = NB // NW
STREAM = 128
CHUNK = 1280
S_PER_CHUNK = CHUNK // STREAM
N_CHUNK = ROWS_PER_W // CHUNK
N_STREAMS = ROWS_PER_W // STREAM
NBUF = 2

_mesh = plsc.VectorSubcoreMesh(core_axis_name="c", subcore_axis_name="s")


@functools.partial(
    pl.kernel,
    out_type=jax.ShapeDtypeStruct((NB, EMB), jnp.float32),
    mesh=_mesh,
    scratch_types=[
        pltpu.VMEM((N_STREAMS, STREAM), jnp.int32),
        pltpu.VMEM((NBUF, CHUNK, EMB), jnp.float32),
        pltpu.SemaphoreType.DMA((NBUF,)),
    ],
    compiler_params=pltpu.CompilerParams(use_tc_tiling_on_sc=False),
)
def _embed_sc(idx_hbm, table_hbm, out_hbm, idx_v, rows_v, sem):
    wid = lax.axis_index("s") * NC + lax.axis_index("c")
    base = wid * ROWS_PER_W

    pltpu.sync_copy(idx_hbm.at[wid], idx_v)

    def fire(c, b):
        for s in range(S_PER_CHUNK):
            pltpu.async_copy(
                table_hbm.at[idx_v.at[c * S_PER_CHUNK + s]],
                rows_v.at[b, pl.ds(s * STREAM, STREAM)],
                sem.at[b],
            )

    def drain(b):
        pltpu.make_async_copy(
            out_hbm.at[pl.ds(0, CHUNK)], rows_v.at[b], sem.at[b]
        ).wait()

    for b in range(NBUF):
        fire(b, b)

    def body(i, _):
        for b in range(NBUF):
            c = i * NBUF + b
            drain(b)
            pltpu.sync_copy(rows_v.at[b], out_hbm.at[pl.ds(base + c * CHUNK, CHUNK)])

            @pl.when(c + NBUF < N_CHUNK)
            def _():
                fire(c + NBUF, b)

        return 0

    lax.fori_loop(0, N_CHUNK // NBUF, body, 0)


def kernel(x, single, emb_weight):
    idx = x.reshape(NW, N_STREAMS, STREAM).astype(jnp.int32)
    out = _embed_sc(idx, emb_weight)
    return out.reshape(B, L, EMB)
